# baseline (speedup 1.0000x reference)
import functools

import jax
import jax.numpy as jnp
from jax import lax
from jax.experimental import pallas as pl
from jax.experimental.pallas import tpu as pltpu
from jax.experimental.pallas import tpu_sc as plsc

B, S, H, I, E, TOPK, TGT = 32, 512, 768, 1536, 8, 2, 128
NC, NS = 2, 16
NW = NC * NS
CHUNK = 64


def _sc_gather_rows(table, ids3):
    nw, nch, chunk = ids3.shape
    n = nw * nch * chunk
    d = table.shape[1]
    mesh = plsc.VectorSubcoreMesh(core_axis_name="c", subcore_axis_name="s")

    @functools.partial(
        pl.kernel,
        mesh=mesh,
        out_type=jax.ShapeDtypeStruct((n, d), table.dtype),
        scratch_types=[
            pltpu.VMEM((nch, chunk), jnp.int32),
            pltpu.VMEM((chunk, d), table.dtype),
            pltpu.VMEM((chunk, d), table.dtype),
            pltpu.SemaphoreType.DMA,
            pltpu.SemaphoreType.DMA,
        ],
    )
    def k(table_hbm, ids_hbm, out_hbm, idx_v, buf0, buf1, gs0, gs1):
        wid = lax.axis_index("s") * NC + lax.axis_index("c")
        pltpu.sync_copy(ids_hbm.at[wid], idx_v)
        bufs = (buf0, buf1)
        gsems = (gs0, gs1)
        base = wid * (nch * chunk)

        def gather(i):
            return pltpu.async_copy(
                table_hbm.at[idx_v.at[i]], bufs[i % 2], gsems[i % 2])

        cps_g = [None] * nch
        cps_g[0] = gather(0)
        for i in range(nch):
            if i + 1 < nch:
                cps_g[i + 1] = gather(i + 1)
            cps_g[i].wait()
            pltpu.sync_copy(
                bufs[i % 2], out_hbm.at[pl.ds(base + i * chunk, chunk)])

    return k(table, ids3)


def _enc_body(x_ref, ids_ref, gate_ref, sgwb_ref, suwb_ref, segw_ref,
              rw_ref, tid_ref, vsum_ref, cls_ref):
    x = x_ref[0]
    logits = lax.dot_general(
        x, gate_ref[...], (((1,), (1,)), ((), ())),
        preferred_element_type=jnp.float32)

    m = jnp.max(logits, axis=0, keepdims=True)
    p = jnp.exp(logits - m)
    p = p / jnp.sum(p, axis=0, keepdims=True)
    sidx = lax.broadcasted_iota(jnp.int32, (S, E), 0)
    v1 = jnp.max(p, axis=0, keepdims=True)
    i1 = jnp.min(jnp.where(p == v1, sidx, S), axis=0, keepdims=True)
    p2 = jnp.where(sidx == i1, -1.0, p)
    v2 = jnp.max(p2, axis=0, keepdims=True)
    i2 = jnp.min(jnp.where(p2 == v2, sidx, S), axis=0, keepdims=True)
    nw1 = v1 / jnp.sum(v1, axis=1, keepdims=True)
    nw2 = v2 / jnp.sum(v2, axis=1, keepdims=True)
    rw_ref[...] = jnp.concatenate([nw1[..., None], nw2[..., None]], axis=-1)
    ids4 = ids_ref[0][..., None]
    pidx = (lax.broadcasted_iota(jnp.int32, (4, 128, 1), 0) * 128
            + lax.broadcasted_iota(jnp.int32, (4, 128, 1), 1))
    t1 = jnp.sum(jnp.where(pidx == i1.reshape(1, 1, E), ids4, 0), axis=(0, 1))
    t2 = jnp.sum(jnp.where(pidx == i2.reshape(1, 1, E), ids4, 0), axis=(0, 1))
    tid_ref[...] = jnp.concatenate([t1[None, :, None], t2[None, :, None]],
                                   axis=-1)

    xb = x.astype(jnp.bfloat16)
    g = lax.dot_general(xb, sgwb_ref[...], (((1,), (1,)), ((), ())),
                        preferred_element_type=jnp.float32)
    u = lax.dot_general(xb, suwb_ref[...], (((1,), (1,)), ((), ())),
                        preferred_element_type=jnp.float32)
    h = (g * jax.nn.sigmoid(g) * u).astype(jnp.bfloat16)
    segate = jax.nn.sigmoid(lax.dot_general(
        x, segw_ref[...], (((1,), (1,)), ((), ())),
        preferred_element_type=jnp.float32))
    vsum_ref[0] = lax.dot_general(segate.astype(jnp.bfloat16), h,
                                  (((0,), (0,)), ((), ())),
                                  preferred_element_type=jnp.float32)
    cls_ref[0] = x[0:1]


def _encoder(hidden, ids3, gate_w, sgwb, suwb, segw):
    nb = hidden.shape[0]
    return pl.pallas_call(
        _enc_body,
        grid=(nb,),
        in_specs=[
            pl.BlockSpec((1, S, H), lambda b: (b, 0, 0)),
            pl.BlockSpec((1, 4, S // 4), lambda b: (b, 0, 0)),
            pl.BlockSpec((E, H), lambda b: (0, 0)),
            pl.BlockSpec((I, H), lambda b: (0, 0)),
            pl.BlockSpec((I, H), lambda b: (0, 0)),
            pl.BlockSpec((1, H), lambda b: (0, 0)),
        ],
        out_specs=[
            pl.BlockSpec((1, E, TOPK), lambda b: (b, 0, 0)),
            pl.BlockSpec((1, E, TOPK), lambda b: (b, 0, 0)),
            pl.BlockSpec((1, 1, I), lambda b: (b, 0, 0)),
            pl.BlockSpec((1, 1, H), lambda b: (b, 0, 0)),
        ],
        out_shape=[
            jax.ShapeDtypeStruct((nb, E, TOPK), jnp.float32),
            jax.ShapeDtypeStruct((nb, E, TOPK), jnp.int32),
            jax.ShapeDtypeStruct((nb, 1, I), jnp.float32),
            jax.ShapeDtypeStruct((nb, 1, H), jnp.float32),
        ],
    )(hidden, ids3, gate_w, sgwb, suwb, segw)


NDISP = TOPK * B


def _expert_body(tid_sref, table_ref, gw_ref, uw_ref, dw_ref, rw_ref,
                 vsum_ref, sdw_ref, cls_ref, fw_ref,
                 fb_ref, ow_ref, ob_ref, out_ref,
                 xbuf_ref, base_ref, sems):
    e = pl.program_id(0)

    def fetch(expert, slot):
        for j in range(NDISP):
            pltpu.make_async_copy(
                table_ref.at[pl.ds(tid_sref[expert * NDISP + j], 1)],
                xbuf_ref.at[slot, pl.ds(j, 1)], sems.at[slot]).start()

    @pl.when(e == 0)
    def _():
        fetch(0, 0)
        shared = lax.dot_general(vsum_ref[...], sdw_ref[...],
                                 (((1,), (1,)), ((), ())),
                                 preferred_element_type=jnp.float32)
        base_ref[...] = (
            lax.dot_general(shared, fw_ref[:, H:2 * H],
                            (((1,), (1,)), ((), ())),
                            preferred_element_type=jnp.float32)
            + lax.dot_general(cls_ref[...], fw_ref[:, 2 * H:],
                              (((1,), (1,)), ((), ())),
                              preferred_element_type=jnp.float32)
            + fb_ref[...]
        )

    @pl.when(e + 1 < E)
    def _():
        fetch(e + 1, (e + 1) % 2)

    slot = e % 2
    pltpu.make_async_copy(
        table_ref.at[pl.ds(0, NDISP)], xbuf_ref.at[slot],
        sems.at[slot]).wait()
    x = xbuf_ref[slot]
    g = lax.dot_general(x, gw_ref[0], (((1,), (1,)), ((), ())),
                        preferred_element_type=jnp.float32)
    u = lax.dot_general(x, uw_ref[0], (((1,), (1,)), ((), ())),
                        preferred_element_type=jnp.float32)
    h = g * jax.nn.sigmoid(g) * u
    hw = h * rw_ref[0][0][:, None]
    v = hw[:B] + hw[B:]
    eo = lax.dot_general(v, dw_ref[0], (((1,), (1,)), ((), ())),
                         preferred_element_type=jnp.float32)
    fh = base_ref[...] + lax.dot_general(eo, fw_ref[:, :H],
                                         (((1,), (1,)), ((), ())),
                                         preferred_element_type=jnp.float32)
    out_ref[0] = lax.dot_general(fh, ow_ref[...], (((1,), (1,)), ((), ())),
                                 preferred_element_type=jnp.float32) + ob_ref[...]


def _experts_head(tid_flat, table, eg, eu, ed, rw, vsum, sdw, cls,
                  fw, fb, ow, ob):
    grid_spec = pltpu.PrefetchScalarGridSpec(
        num_scalar_prefetch=1,
        grid=(E,),
        in_specs=[
            pl.BlockSpec(memory_space=pl.ANY),
            pl.BlockSpec((1, I, H), lambda e, sref: (e, 0, 0)),
            pl.BlockSpec((1, I, H), lambda e, sref: (e, 0, 0)),
            pl.BlockSpec((1, H, I), lambda e, sref: (e, 0, 0)),
            pl.BlockSpec((1, 1, NDISP), lambda e, sref: (e, 0, 0)),
            pl.BlockSpec((B, I), lambda e, sref: (0, 0)),
            pl.BlockSpec((H, I), lambda e, sref: (0, 0)),
            pl.BlockSpec((B, H), lambda e, sref: (0, 0)),
            pl.BlockSpec((H, 3 * H), lambda e, sref: (0, 0)),
            pl.BlockSpec((1, H), lambda e, sref: (0, 0)),
            pl.BlockSpec((TGT, H), lambda e, sref: (0, 0)),
            pl.BlockSpec((1, TGT), lambda e, sref: (0, 0)),
        ],
        out_specs=pl.BlockSpec((1, B, TGT), lambda e, sref: (e, 0, 0)),
        scratch_shapes=[
            pltpu.VMEM((2, NDISP, H), jnp.float32),
            pltpu.VMEM((B, H), jnp.float32),
            pltpu.SemaphoreType.DMA((2,)),
        ],
    )
    return pl.pallas_call(
        _expert_body,
        grid_spec=grid_spec,
        out_shape=jax.ShapeDtypeStruct((E, B, TGT), jnp.float32),
    )(tid_flat, table, eg, eu, ed, rw, vsum, sdw, cls, fw, fb, ow, ob)


def kernel(input_ids, token_type_ids, attention_mask, embed_table, gate_w,
           expert_gate, expert_up, expert_down,
           shared_gate_w, shared_up_w, shared_down_w, shared_expert_gate_w,
           feature_w, feature_b, output_w, output_b):
    del token_type_ids, attention_mask
    ids = input_ids.reshape(-1).astype(jnp.int32)
    sgwb = shared_gate_w.astype(jnp.bfloat16)
    suwb = shared_up_w.astype(jnp.bfloat16)
    group_sizes = (8, 24)
    parts = []
    start = 0
    for bg in group_sizes:
        rows_g = bg * S
        ids_g = lax.slice(ids, (start * S,), (start * S + rows_g,))
        hid_g = _sc_gather_rows(
            embed_table, ids_g.reshape(NW, rows_g // (NW * CHUNK), CHUNK))
        parts.append(_encoder(
            hid_g.reshape(bg, S, H), ids_g.reshape(bg, 4, S // 4), gate_w,
            sgwb, suwb, shared_expert_gate_w))
        start += bg
    rw = jnp.concatenate([p[0] for p in parts], axis=0)
    tid = jnp.concatenate([p[1] for p in parts], axis=0)
    vsum = jnp.concatenate([p[2] for p in parts], axis=0).reshape(B, I)
    cls = jnp.concatenate([p[3] for p in parts], axis=0).reshape(B, H)

    tid_flat = tid.transpose(1, 2, 0).reshape(-1)
    rw_ekb = rw.transpose(1, 2, 0).reshape(E, 1, NDISP)

    out = _experts_head(
        tid_flat, embed_table, expert_gate, expert_up, expert_down, rw_ekb,
        vsum, shared_down_w, cls, feature_w,
        feature_b.reshape(1, H), output_w, output_b.reshape(1, TGT))
    return out.transpose(1, 0, 2)


# speedup vs baseline: 1.0264x; 1.0264x over previous
import functools

import jax
import jax.numpy as jnp
from jax import lax
from jax.experimental import pallas as pl
from jax.experimental.pallas import tpu as pltpu
from jax.experimental.pallas import tpu_sc as plsc

B, S, H, I, E, TOPK, TGT = 32, 512, 768, 1536, 8, 2, 128
NC, NS = 2, 16
NW = NC * NS
CHUNK = 64


def _sc_gather_rows(table, ids3):
    nw, nch, chunk = ids3.shape
    n = nw * nch * chunk
    d = table.shape[1]
    mesh = plsc.VectorSubcoreMesh(core_axis_name="c", subcore_axis_name="s")

    @functools.partial(
        pl.kernel,
        mesh=mesh,
        out_type=jax.ShapeDtypeStruct((n, d), table.dtype),
        scratch_types=[
            pltpu.VMEM((nch, chunk), jnp.int32),
            pltpu.VMEM((chunk, d), table.dtype),
            pltpu.VMEM((chunk, d), table.dtype),
            pltpu.SemaphoreType.DMA,
            pltpu.SemaphoreType.DMA,
        ],
    )
    def k(table_hbm, ids_hbm, out_hbm, idx_v, buf0, buf1, gs0, gs1):
        wid = lax.axis_index("s") * NC + lax.axis_index("c")
        pltpu.sync_copy(ids_hbm.at[wid], idx_v)
        bufs = (buf0, buf1)
        gsems = (gs0, gs1)
        base = wid * (nch * chunk)

        def gather(i):
            return pltpu.async_copy(
                table_hbm.at[idx_v.at[i]], bufs[i % 2], gsems[i % 2])

        cps_g = [None] * nch
        cps_g[0] = gather(0)
        for i in range(nch):
            if i + 1 < nch:
                cps_g[i + 1] = gather(i + 1)
            cps_g[i].wait()
            pltpu.sync_copy(
                bufs[i % 2], out_hbm.at[pl.ds(base + i * chunk, chunk)])

    return k(table, ids3)


def _enc_body(x_ref, ids_ref, gate_ref, sgwb_ref, suwb_ref, segw_ref,
              rw_ref, tid_ref, vsum_ref, cls_ref):
    x = x_ref[0]
    logits = lax.dot_general(
        x, gate_ref[...], (((1,), (1,)), ((), ())),
        preferred_element_type=jnp.float32)

    m = jnp.max(logits, axis=0, keepdims=True)
    p = jnp.exp(logits - m)
    p = p / jnp.sum(p, axis=0, keepdims=True)
    sidx = lax.broadcasted_iota(jnp.int32, (S, E), 0)
    v1 = jnp.max(p, axis=0, keepdims=True)
    i1 = jnp.min(jnp.where(p == v1, sidx, S), axis=0, keepdims=True)
    p2 = jnp.where(sidx == i1, -1.0, p)
    v2 = jnp.max(p2, axis=0, keepdims=True)
    i2 = jnp.min(jnp.where(p2 == v2, sidx, S), axis=0, keepdims=True)
    nw1 = v1 / jnp.sum(v1, axis=1, keepdims=True)
    nw2 = v2 / jnp.sum(v2, axis=1, keepdims=True)
    rw_ref[...] = jnp.concatenate([nw1[..., None], nw2[..., None]], axis=-1)
    ids4 = ids_ref[0][..., None]
    pidx = (lax.broadcasted_iota(jnp.int32, (4, 128, 1), 0) * 128
            + lax.broadcasted_iota(jnp.int32, (4, 128, 1), 1))
    t1 = jnp.sum(jnp.where(pidx == i1.reshape(1, 1, E), ids4, 0), axis=(0, 1))
    t2 = jnp.sum(jnp.where(pidx == i2.reshape(1, 1, E), ids4, 0), axis=(0, 1))
    tid_ref[...] = jnp.concatenate([t1[None, :, None], t2[None, :, None]],
                                   axis=-1)

    xb = x.astype(jnp.bfloat16)
    g = lax.dot_general(xb, sgwb_ref[...], (((1,), (1,)), ((), ())),
                        preferred_element_type=jnp.float32)
    u = lax.dot_general(xb, suwb_ref[...], (((1,), (1,)), ((), ())),
                        preferred_element_type=jnp.float32)
    h = (g * jax.nn.sigmoid(g) * u).astype(jnp.bfloat16)
    segate = jax.nn.sigmoid(lax.dot_general(
        x, segw_ref[...], (((1,), (1,)), ((), ())),
        preferred_element_type=jnp.float32))
    vsum_ref[0] = lax.dot_general(segate.astype(jnp.bfloat16), h,
                                  (((0,), (0,)), ((), ())),
                                  preferred_element_type=jnp.float32)
    cls_ref[0] = x[0:1]


def _encoder(hidden, ids3, gate_w, sgwb, suwb, segw):
    nb = hidden.shape[0]
    return pl.pallas_call(
        _enc_body,
        grid=(nb,),
        in_specs=[
            pl.BlockSpec((1, S, H), lambda b: (b, 0, 0)),
            pl.BlockSpec((1, 4, S // 4), lambda b: (b, 0, 0)),
            pl.BlockSpec((E, H), lambda b: (0, 0)),
            pl.BlockSpec((I, H), lambda b: (0, 0)),
            pl.BlockSpec((I, H), lambda b: (0, 0)),
            pl.BlockSpec((1, H), lambda b: (0, 0)),
        ],
        out_specs=[
            pl.BlockSpec((1, E, TOPK), lambda b: (b, 0, 0)),
            pl.BlockSpec((1, E, TOPK), lambda b: (b, 0, 0)),
            pl.BlockSpec((1, 1, I), lambda b: (b, 0, 0)),
            pl.BlockSpec((1, 1, H), lambda b: (b, 0, 0)),
        ],
        out_shape=[
            jax.ShapeDtypeStruct((nb, E, TOPK), jnp.float32),
            jax.ShapeDtypeStruct((nb, E, TOPK), jnp.int32),
            jax.ShapeDtypeStruct((nb, 1, I), jnp.float32),
            jax.ShapeDtypeStruct((nb, 1, H), jnp.float32),
        ],
    )(hidden, ids3, gate_w, sgwb, suwb, segw)


NDISP = TOPK * B


def _expert_body(tid_sref, table_ref, gw_ref, uw_ref, dw_ref, rw_ref,
                 vsum_ref, sdw_ref, cls_ref, fw_ref,
                 fb_ref, ow_ref, ob_ref, out_ref,
                 xbuf_ref, base_ref, sems):
    e = pl.program_id(0)

    def fetch(expert, slot):
        for j in range(NDISP):
            pltpu.make_async_copy(
                table_ref.at[pl.ds(tid_sref[expert * NDISP + j], 1)],
                xbuf_ref.at[slot, pl.ds(j, 1)], sems.at[slot]).start()

    @pl.when(e == 0)
    def _():
        fetch(0, 0)
        shared = lax.dot_general(vsum_ref[...], sdw_ref[...],
                                 (((1,), (1,)), ((), ())),
                                 preferred_element_type=jnp.float32)
        base_ref[...] = (
            lax.dot_general(shared, fw_ref[:, H:2 * H],
                            (((1,), (1,)), ((), ())),
                            preferred_element_type=jnp.float32)
            + lax.dot_general(cls_ref[...], fw_ref[:, 2 * H:],
                              (((1,), (1,)), ((), ())),
                              preferred_element_type=jnp.float32)
            + fb_ref[...]
        )

    @pl.when(e + 1 < E)
    def _():
        fetch(e + 1, (e + 1) % 2)

    slot = e % 2
    pltpu.make_async_copy(
        table_ref.at[pl.ds(0, NDISP)], xbuf_ref.at[slot],
        sems.at[slot]).wait()
    x = xbuf_ref[slot]
    g = lax.dot_general(x, gw_ref[0], (((1,), (1,)), ((), ())),
                        preferred_element_type=jnp.float32)
    u = lax.dot_general(x, uw_ref[0], (((1,), (1,)), ((), ())),
                        preferred_element_type=jnp.float32)
    h = g * jax.nn.sigmoid(g) * u
    hw = h * rw_ref[0][0][:, None]
    v = hw[:B] + hw[B:]
    eo = lax.dot_general(v, dw_ref[0], (((1,), (1,)), ((), ())),
                         preferred_element_type=jnp.float32)
    fh = base_ref[...] + lax.dot_general(eo, fw_ref[:, :H],
                                         (((1,), (1,)), ((), ())),
                                         preferred_element_type=jnp.float32)
    out_ref[0] = lax.dot_general(fh, ow_ref[...], (((1,), (1,)), ((), ())),
                                 preferred_element_type=jnp.float32) + ob_ref[...]


def _experts_head(tid_flat, table, eg, eu, ed, rw, vsum, sdw, cls,
                  fw, fb, ow, ob):
    grid_spec = pltpu.PrefetchScalarGridSpec(
        num_scalar_prefetch=1,
        grid=(E,),
        in_specs=[
            pl.BlockSpec(memory_space=pl.ANY),
            pl.BlockSpec((1, I, H), lambda e, sref: (e, 0, 0)),
            pl.BlockSpec((1, I, H), lambda e, sref: (e, 0, 0)),
            pl.BlockSpec((1, H, I), lambda e, sref: (e, 0, 0)),
            pl.BlockSpec((1, 1, NDISP), lambda e, sref: (e, 0, 0)),
            pl.BlockSpec((B, I), lambda e, sref: (0, 0)),
            pl.BlockSpec((H, I), lambda e, sref: (0, 0)),
            pl.BlockSpec((B, H), lambda e, sref: (0, 0)),
            pl.BlockSpec((H, 3 * H), lambda e, sref: (0, 0)),
            pl.BlockSpec((1, H), lambda e, sref: (0, 0)),
            pl.BlockSpec((TGT, H), lambda e, sref: (0, 0)),
            pl.BlockSpec((1, TGT), lambda e, sref: (0, 0)),
        ],
        out_specs=pl.BlockSpec((1, B, TGT), lambda e, sref: (e, 0, 0)),
        scratch_shapes=[
            pltpu.VMEM((2, NDISP, H), jnp.float32),
            pltpu.VMEM((B, H), jnp.float32),
            pltpu.SemaphoreType.DMA((2,)),
        ],
    )
    return pl.pallas_call(
        _expert_body,
        grid_spec=grid_spec,
        out_shape=jax.ShapeDtypeStruct((E, B, TGT), jnp.float32),
    )(tid_flat, table, eg, eu, ed, rw, vsum, sdw, cls, fw, fb, ow, ob)


def kernel(input_ids, token_type_ids, attention_mask, embed_table, gate_w,
           expert_gate, expert_up, expert_down,
           shared_gate_w, shared_up_w, shared_down_w, shared_expert_gate_w,
           feature_w, feature_b, output_w, output_b):
    del token_type_ids, attention_mask
    ids = input_ids.reshape(-1).astype(jnp.int32)
    sgwb = shared_gate_w.astype(jnp.bfloat16)
    suwb = shared_up_w.astype(jnp.bfloat16)
    group_sizes = (16, 16)
    parts = []
    start = 0
    for bg in group_sizes:
        rows_g = bg * S
        ids_g = lax.slice(ids, (start * S,), (start * S + rows_g,))
        hid_g = _sc_gather_rows(
            embed_table, ids_g.reshape(NW, rows_g // (NW * CHUNK), CHUNK))
        parts.append(_encoder(
            hid_g.reshape(bg, S, H), ids_g.reshape(bg, 4, S // 4), gate_w,
            sgwb, suwb, shared_expert_gate_w))
        start += bg
    rw = jnp.concatenate([p[0] for p in parts], axis=0)
    tid = jnp.concatenate([p[1] for p in parts], axis=0)
    vsum = jnp.concatenate([p[2] for p in parts], axis=0).reshape(B, I)
    cls = jnp.concatenate([p[3] for p in parts], axis=0).reshape(B, H)

    tid_flat = tid.transpose(1, 2, 0).reshape(-1)
    rw_ekb = rw.transpose(1, 2, 0).reshape(E, 1, NDISP)

    out = _experts_head(
        tid_flat, embed_table, expert_gate, expert_up, expert_down, rw_ekb,
        vsum, shared_down_w, cls, feature_w,
        feature_b.reshape(1, H), output_w, output_b.reshape(1, TGT))
    return out.transpose(1, 0, 2)


# speedup vs baseline: 1.0296x; 1.0032x over previous
import functools

import jax
import jax.numpy as jnp
from jax import lax
from jax.experimental import pallas as pl
from jax.experimental.pallas import tpu as pltpu
from jax.experimental.pallas import tpu_sc as plsc

B, S, H, I, E, TOPK, TGT = 32, 512, 768, 1536, 8, 2, 128
NC, NS = 2, 16
NW = NC * NS
CHUNK = 32


def _sc_gather_rows(table, ids3):
    nw, nch, chunk = ids3.shape
    n = nw * nch * chunk
    d = table.shape[1]
    mesh = plsc.VectorSubcoreMesh(core_axis_name="c", subcore_axis_name="s")

    @functools.partial(
        pl.kernel,
        mesh=mesh,
        out_type=jax.ShapeDtypeStruct((n, d), table.dtype),
        scratch_types=[
            pltpu.VMEM((nch, chunk), jnp.int32),
            pltpu.VMEM((4, chunk, d), table.dtype),
            pltpu.SemaphoreType.DMA((4,)),
            pltpu.SemaphoreType.DMA((4,)),
        ],
    )
    def k(table_hbm, ids_hbm, out_hbm, idx_v, bufs, gsems, osems):
        wid = lax.axis_index("s") * NC + lax.axis_index("c")
        pltpu.sync_copy(ids_hbm.at[wid], idx_v)
        base = wid * (nch * chunk)

        def gather(i):
            return pltpu.async_copy(
                table_hbm.at[idx_v.at[i]], bufs.at[i % 4], gsems.at[i % 4])

        def put(i):
            return pltpu.async_copy(
                bufs.at[i % 4], out_hbm.at[pl.ds(base + i * chunk, chunk)],
                osems.at[i % 4])

        cps_g = [None] * nch
        cps_o = [None] * nch
        for j in range(min(4, nch)):
            cps_g[j] = gather(j)
        for i in range(nch):
            cps_g[i].wait()
            cps_o[i] = put(i)
            if i >= 1 and i + 3 < nch:
                cps_o[i - 1].wait()
                cps_g[i + 3] = gather(i + 3)
        for j in range(max(0, nch - 4), nch):
            cps_o[j].wait()

    return k(table, ids3)


def _enc_body(x_ref, ids_ref, gate_ref, sgwb_ref, suwb_ref, segw_ref,
              rw_ref, tid_ref, vsum_ref, cls_ref):
    x = x_ref[0]
    logits = lax.dot_general(
        x, gate_ref[...], (((1,), (1,)), ((), ())),
        preferred_element_type=jnp.float32)

    m = jnp.max(logits, axis=0, keepdims=True)
    p = jnp.exp(logits - m)
    p = p / jnp.sum(p, axis=0, keepdims=True)
    sidx = lax.broadcasted_iota(jnp.int32, (S, E), 0)
    v1 = jnp.max(p, axis=0, keepdims=True)
    i1 = jnp.min(jnp.where(p == v1, sidx, S), axis=0, keepdims=True)
    p2 = jnp.where(sidx == i1, -1.0, p)
    v2 = jnp.max(p2, axis=0, keepdims=True)
    i2 = jnp.min(jnp.where(p2 == v2, sidx, S), axis=0, keepdims=True)
    nw1 = v1 / jnp.sum(v1, axis=1, keepdims=True)
    nw2 = v2 / jnp.sum(v2, axis=1, keepdims=True)
    rw_ref[...] = jnp.concatenate([nw1[..., None], nw2[..., None]], axis=-1)
    ids4 = ids_ref[0][..., None]
    pidx = (lax.broadcasted_iota(jnp.int32, (4, 128, 1), 0) * 128
            + lax.broadcasted_iota(jnp.int32, (4, 128, 1), 1))
    t1 = jnp.sum(jnp.where(pidx == i1.reshape(1, 1, E), ids4, 0), axis=(0, 1))
    t2 = jnp.sum(jnp.where(pidx == i2.reshape(1, 1, E), ids4, 0), axis=(0, 1))
    tid_ref[...] = jnp.concatenate([t1[None, :, None], t2[None, :, None]],
                                   axis=-1)

    xb = x.astype(jnp.bfloat16)
    g = lax.dot_general(xb, sgwb_ref[...], (((1,), (1,)), ((), ())),
                        preferred_element_type=jnp.float32)
    u = lax.dot_general(xb, suwb_ref[...], (((1,), (1,)), ((), ())),
                        preferred_element_type=jnp.float32)
    h = (g * jax.nn.sigmoid(g) * u).astype(jnp.bfloat16)
    segate = jax.nn.sigmoid(lax.dot_general(
        x, segw_ref[...], (((1,), (1,)), ((), ())),
        preferred_element_type=jnp.float32))
    vsum_ref[0] = lax.dot_general(segate.astype(jnp.bfloat16), h,
                                  (((0,), (0,)), ((), ())),
                                  preferred_element_type=jnp.float32)
    cls_ref[0] = x[0:1]


def _encoder(hidden, ids3, gate_w, sgwb, suwb, segw):
    nb = hidden.shape[0]
    return pl.pallas_call(
        _enc_body,
        grid=(nb,),
        in_specs=[
            pl.BlockSpec((1, S, H), lambda b: (b, 0, 0)),
            pl.BlockSpec((1, 4, S // 4), lambda b: (b, 0, 0)),
            pl.BlockSpec((E, H), lambda b: (0, 0)),
            pl.BlockSpec((I, H), lambda b: (0, 0)),
            pl.BlockSpec((I, H), lambda b: (0, 0)),
            pl.BlockSpec((1, H), lambda b: (0, 0)),
        ],
        out_specs=[
            pl.BlockSpec((1, E, TOPK), lambda b: (b, 0, 0)),
            pl.BlockSpec((1, E, TOPK), lambda b: (b, 0, 0)),
            pl.BlockSpec((1, 1, I), lambda b: (b, 0, 0)),
            pl.BlockSpec((1, 1, H), lambda b: (b, 0, 0)),
        ],
        out_shape=[
            jax.ShapeDtypeStruct((nb, E, TOPK), jnp.float32),
            jax.ShapeDtypeStruct((nb, E, TOPK), jnp.int32),
            jax.ShapeDtypeStruct((nb, 1, I), jnp.float32),
            jax.ShapeDtypeStruct((nb, 1, H), jnp.float32),
        ],
    )(hidden, ids3, gate_w, sgwb, suwb, segw)


NDISP = TOPK * B


def _expert_body(tid_sref, table_ref, gw_ref, uw_ref, dw_ref, rw_ref,
                 vsum_ref, sdw_ref, cls_ref, fw_ref,
                 fb_ref, ow_ref, ob_ref, out_ref,
                 xbuf_ref, base_ref, sems):
    e = pl.program_id(0)

    def fetch(expert, slot):
        for j in range(NDISP):
            pltpu.make_async_copy(
                table_ref.at[pl.ds(tid_sref[expert * NDISP + j], 1)],
                xbuf_ref.at[slot, pl.ds(j, 1)], sems.at[slot]).start()

    @pl.when(e == 0)
    def _():
        fetch(0, 0)
        shared = lax.dot_general(vsum_ref[...], sdw_ref[...],
                                 (((1,), (1,)), ((), ())),
                                 preferred_element_type=jnp.float32)
        base_ref[...] = (
            lax.dot_general(shared, fw_ref[:, H:2 * H],
                            (((1,), (1,)), ((), ())),
                            preferred_element_type=jnp.float32)
            + lax.dot_general(cls_ref[...], fw_ref[:, 2 * H:],
                              (((1,), (1,)), ((), ())),
                              preferred_element_type=jnp.float32)
            + fb_ref[...]
        )

    @pl.when(e + 1 < E)
    def _():
        fetch(e + 1, (e + 1) % 2)

    slot = e % 2
    pltpu.make_async_copy(
        table_ref.at[pl.ds(0, NDISP)], xbuf_ref.at[slot],
        sems.at[slot]).wait()
    x = xbuf_ref[slot]
    g = lax.dot_general(x, gw_ref[0], (((1,), (1,)), ((), ())),
                        preferred_element_type=jnp.float32)
    u = lax.dot_general(x, uw_ref[0], (((1,), (1,)), ((), ())),
                        preferred_element_type=jnp.float32)
    h = g * jax.nn.sigmoid(g) * u
    hw = h * rw_ref[0][0][:, None]
    v = hw[:B] + hw[B:]
    eo = lax.dot_general(v, dw_ref[0], (((1,), (1,)), ((), ())),
                         preferred_element_type=jnp.float32)
    fh = base_ref[...] + lax.dot_general(eo, fw_ref[:, :H],
                                         (((1,), (1,)), ((), ())),
                                         preferred_element_type=jnp.float32)
    out_ref[0] = lax.dot_general(fh, ow_ref[...], (((1,), (1,)), ((), ())),
                                 preferred_element_type=jnp.float32) + ob_ref[...]


def _experts_head(tid_flat, table, eg, eu, ed, rw, vsum, sdw, cls,
                  fw, fb, ow, ob):
    grid_spec = pltpu.PrefetchScalarGridSpec(
        num_scalar_prefetch=1,
        grid=(E,),
        in_specs=[
            pl.BlockSpec(memory_space=pl.ANY),
            pl.BlockSpec((1, I, H), lambda e, sref: (e, 0, 0)),
            pl.BlockSpec((1, I, H), lambda e, sref: (e, 0, 0)),
            pl.BlockSpec((1, H, I), lambda e, sref: (e, 0, 0)),
            pl.BlockSpec((1, 1, NDISP), lambda e, sref: (e, 0, 0)),
            pl.BlockSpec((B, I), lambda e, sref: (0, 0)),
            pl.BlockSpec((H, I), lambda e, sref: (0, 0)),
            pl.BlockSpec((B, H), lambda e, sref: (0, 0)),
            pl.BlockSpec((H, 3 * H), lambda e, sref: (0, 0)),
            pl.BlockSpec((1, H), lambda e, sref: (0, 0)),
            pl.BlockSpec((TGT, H), lambda e, sref: (0, 0)),
            pl.BlockSpec((1, TGT), lambda e, sref: (0, 0)),
        ],
        out_specs=pl.BlockSpec((1, B, TGT), lambda e, sref: (e, 0, 0)),
        scratch_shapes=[
            pltpu.VMEM((2, NDISP, H), jnp.float32),
            pltpu.VMEM((B, H), jnp.float32),
            pltpu.SemaphoreType.DMA((2,)),
        ],
    )
    return pl.pallas_call(
        _expert_body,
        grid_spec=grid_spec,
        out_shape=jax.ShapeDtypeStruct((E, B, TGT), jnp.float32),
    )(tid_flat, table, eg, eu, ed, rw, vsum, sdw, cls, fw, fb, ow, ob)


def kernel(input_ids, token_type_ids, attention_mask, embed_table, gate_w,
           expert_gate, expert_up, expert_down,
           shared_gate_w, shared_up_w, shared_down_w, shared_expert_gate_w,
           feature_w, feature_b, output_w, output_b):
    del token_type_ids, attention_mask
    ids = input_ids.reshape(-1).astype(jnp.int32)
    sgwb = shared_gate_w.astype(jnp.bfloat16)
    suwb = shared_up_w.astype(jnp.bfloat16)
    group_sizes = (16, 16)
    parts = []
    start = 0
    for bg in group_sizes:
        rows_g = bg * S
        ids_g = lax.slice(ids, (start * S,), (start * S + rows_g,))
        hid_g = _sc_gather_rows(
            embed_table, ids_g.reshape(NW, rows_g // (NW * CHUNK), CHUNK))
        parts.append(_encoder(
            hid_g.reshape(bg, S, H), ids_g.reshape(bg, 4, S // 4), gate_w,
            sgwb, suwb, shared_expert_gate_w))
        start += bg
    rw = jnp.concatenate([p[0] for p in parts], axis=0)
    tid = jnp.concatenate([p[1] for p in parts], axis=0)
    vsum = jnp.concatenate([p[2] for p in parts], axis=0).reshape(B, I)
    cls = jnp.concatenate([p[3] for p in parts], axis=0).reshape(B, H)

    tid_flat = tid.transpose(1, 2, 0).reshape(-1)
    rw_ekb = rw.transpose(1, 2, 0).reshape(E, 1, NDISP)

    out = _experts_head(
        tid_flat, embed_table, expert_gate, expert_up, expert_down, rw_ekb,
        vsum, shared_down_w, cls, feature_w,
        feature_b.reshape(1, H), output_w, output_b.reshape(1, TGT))
    return out.transpose(1, 0, 2)
